# baseline (speedup 1.0000x reference)
import jax
import jax.numpy as jnp
from jax import lax
from jax.experimental import pallas as pl
from jax.experimental.pallas import tpu as pltpu
from jax.experimental.pallas import tpu_sc as plsc

B = 16384
C = 11
HW = 64
P = 32
OUT = 64
L = 16
NC = 2
NS = 16
NW = NC * NS
S = 4
BS = B // S
WB = BS // NW
NH = WB // L


def _make_sc_body(s):
    soff = s * BS

    def _sc_extract(ids_hbm, board_hbm, raw_hbm,
                    bb0, bb1, fidxb, ob0, ob1, si0, si1, so0, so1):
        cid = lax.axis_index("c")
        sid = lax.axis_index("s")
        wid = sid * NC + cid
        col0 = wid * WB
        hcol0 = soff + col0
        lane = lax.iota(jnp.int32, L)
        zero16 = jnp.zeros((L,), jnp.float32)
        sent = jnp.full((L,), HW, jnp.int32)

        bbs = (bb0, bb1)
        obs = (ob0, ob1)
        sis = (si0, si1)
        sos = (so0, so1)

        def in_dma(c, buf):
            return pltpu.make_async_copy(
                board_hbm.at[pl.ds(c * HW, HW), pl.ds(hcol0, WB)],
                bbs[buf].at[pl.ds(0, HW)], sis[buf])

        def out_dma(c, buf):
            return pltpu.make_async_copy(
                obs[buf], raw_hbm.at[:, pl.ds(c, 1), pl.ds(col0, WB)],
                sos[buf])

        @plsc.parallel_loop(0, NH)
        def _init(h):
            hh = h * L
            bb0[HW, pl.ds(hh, L)] = zero16
            bb1[HW, pl.ds(hh, L)] = zero16
            for p in range(P + 1):
                fidxb[p, pl.ds(hh, L)] = sent

        pltpu.sync_copy(ids_hbm.at[:, pl.ds(hcol0, WB)],
                        bb0.at[pl.ds(0, HW)])
        in_dma(0, 1).start()

        @plsc.parallel_loop(0, NH)
        def _scan(h):
            hh = h * L
            lv = lane + hh
            for hw in range(HW - 1, -1, -1):
                idv = plsc.bitcast(bb0[hw, pl.ds(hh, L)], jnp.int32)
                plsc.store_scatter(fidxb, [idv, lv],
                                   jnp.full((L,), hw, jnp.int32))

        def gather_pass(buf, obuf):
            src = bbs[buf]
            dst = obs[obuf]

            @plsc.parallel_loop(0, NH)
            def _gather(h):
                hh = h * L
                lv = lane + hh
                for p in range(P):
                    fv = fidxb[p + 1, pl.ds(hh, L)]
                    gv = plsc.load_gather(src, [fv, lv])
                    dst[p, 0, pl.ds(hh, L)] = gv

        def pair(k, _):
            c0 = 2 * k
            in_dma(c0, 1).wait()

            @pl.when(k > 0)
            def _():
                out_dma(c0 - 2, 0).wait()
            in_dma(c0 + 1, 0).start()
            gather_pass(1, 0)
            out_dma(c0, 0).start()

            c1 = 2 * k + 1
            in_dma(c1, 0).wait()

            @pl.when(k > 0)
            def _():
                out_dma(c1 - 2, 1).wait()
            in_dma(c1 + 1, 1).start()
            gather_pass(0, 1)
            out_dma(c1, 1).start()
            return 0
        lax.fori_loop(0, (C - 1) // 2, pair, 0)

        in_dma(C - 1, 1).wait()
        out_dma(C - 3, 0).wait()
        gather_pass(1, 0)
        out_dma(C - 1, 0).start()

        out_dma(C - 2, 1).wait()

        @plsc.parallel_loop(0, NH)
        def _fill0(h):
            for p in range(P):
                ob1[p, 0, pl.ds(h * L, L)] = zero16

        for c in range(C, L - 1):
            pltpu.sync_copy(ob1, raw_hbm.at[:, pl.ds(c, 1), pl.ds(col0, WB)])

        out_dma(C - 1, 0).wait()

        @plsc.parallel_loop(0, NH)
        def _fill1(h):
            for p in range(P):
                ob0[p, 0, pl.ds(h * L, L)] = zero16 + 1.0

        pltpu.sync_copy(ob0, raw_hbm.at[:, pl.ds(L - 1, 1), pl.ds(col0, WB)])

    return _sc_extract


def _tc_project(w_ref, x_ref, o_ref):
    o_ref[...] = jnp.dot(w_ref[...], x_ref[...],
                         preferred_element_type=jnp.float32)


def _tc_project_alias(w_ref, x_ref, y_ref, o_ref):
    del y_ref
    o_ref[...] = jnp.dot(w_ref[...], x_ref[...],
                         preferred_element_type=jnp.float32)


def kernel(full_board_vector, piece_ids, proj_w, proj_b):
    ids_t = jax.lax.bitcast_convert_type(
        piece_ids.transpose(1, 2, 0).reshape(HW, B), jnp.float32)
    board_t = full_board_vector.transpose(1, 2, 3, 0).reshape(C * HW, B)

    sc_scratch = [
        pltpu.VMEM((HW + 1, WB), jnp.float32),
        pltpu.VMEM((HW + 1, WB), jnp.float32),
        pltpu.VMEM((P + 1, WB), jnp.int32),
        pltpu.VMEM((P, 1, WB), jnp.float32),
        pltpu.VMEM((P, 1, WB), jnp.float32),
        pltpu.SemaphoreType.DMA,
        pltpu.SemaphoreType.DMA,
        pltpu.SemaphoreType.DMA,
        pltpu.SemaphoreType.DMA,
    ]
    mesh = plsc.VectorSubcoreMesh(core_axis_name="c", subcore_axis_name="s",
                                  num_cores=NC, num_subcores=NS)
    raws = []
    for s in range(S):
        sc_call = pl.kernel(
            _make_sc_body(s),
            out_type=jax.ShapeDtypeStruct((P, L, BS), jnp.float32),
            mesh=mesh,
            compiler_params=pltpu.CompilerParams(needs_layout_passes=False),
            scratch_types=sc_scratch,
        )
        raws.append(sc_call(ids_t, board_t).reshape(P * L, BS))

    wb = jnp.zeros((OUT, L), jnp.float32).at[:, :C].set(proj_w)
    wb = wb.at[:, L - 1].set(proj_b)
    w16 = jnp.kron(jnp.eye(L, dtype=jnp.float32), wb)

    NB = 2048
    nbs = BS // NB
    y = None
    for s in range(S):
        w_spec = pl.BlockSpec((L * OUT, L * L), lambda g, nb: (0, 0))
        x_spec = pl.BlockSpec((L * L, NB), lambda g, nb: (g, nb))
        o_spec = pl.BlockSpec((L * OUT, NB),
                              lambda g, nb, s=s: (g, nb + s * nbs))
        if y is None:
            y = pl.pallas_call(
                _tc_project,
                grid=(2, nbs),
                in_specs=[w_spec, x_spec],
                out_specs=o_spec,
                out_shape=jax.ShapeDtypeStruct((P * OUT, B), jnp.float32),
            )(w16, raws[s])
        else:
            y = pl.pallas_call(
                _tc_project_alias,
                grid=(2, nbs),
                in_specs=[w_spec, x_spec,
                          pl.BlockSpec(memory_space=pl.ANY)],
                out_specs=o_spec,
                out_shape=jax.ShapeDtypeStruct((P * OUT, B), jnp.float32),
                input_output_aliases={2: 0},
            )(w16, raws[s], y)
    return y.reshape(P, OUT, B).transpose(2, 0, 1)


# speedup vs baseline: 1.1914x; 1.1914x over previous
import jax
import jax.numpy as jnp
from jax import lax
from jax.experimental import pallas as pl
from jax.experimental.pallas import tpu as pltpu
from jax.experimental.pallas import tpu_sc as plsc

B = 16384
C = 11
HW = 64
P = 32
OUT = 64
L = 16
NC = 2
NS = 16
NW = NC * NS
S = 2
BS = B // S
WB = BS // NW
NH = WB // L


def _make_sc_body(s):
    soff = s * BS

    def _sc_extract(ids_hbm, board_hbm, raw_hbm,
                    bb0, bb1, fidxb, ob0, ob1, si0, si1, so0, so1):
        cid = lax.axis_index("c")
        sid = lax.axis_index("s")
        wid = sid * NC + cid
        col0 = wid * WB
        hcol0 = soff + col0
        lane = lax.iota(jnp.int32, L)
        zero16 = jnp.zeros((L,), jnp.float32)
        sent = jnp.full((L,), HW, jnp.int32)

        bbs = (bb0, bb1)
        obs = (ob0, ob1)
        sis = (si0, si1)
        sos = (so0, so1)

        def in_dma(c, buf):
            return pltpu.make_async_copy(
                board_hbm.at[pl.ds(c * HW, HW), pl.ds(hcol0, WB)],
                bbs[buf].at[pl.ds(0, HW)], sis[buf])

        def out_dma(c, buf):
            return pltpu.make_async_copy(
                obs[buf], raw_hbm.at[:, pl.ds(c, 1), pl.ds(col0, WB)],
                sos[buf])

        @plsc.parallel_loop(0, NH)
        def _init(h):
            hh = h * L
            bb0[HW, pl.ds(hh, L)] = zero16
            bb1[HW, pl.ds(hh, L)] = zero16
            for p in range(P + 1):
                fidxb[p, pl.ds(hh, L)] = sent

        pltpu.sync_copy(ids_hbm.at[:, pl.ds(hcol0, WB)],
                        bb0.at[pl.ds(0, HW)])
        in_dma(0, 1).start()

        @plsc.parallel_loop(0, NH)
        def _scan(h):
            hh = h * L
            lv = lane + hh
            for hw in range(HW - 1, -1, -1):
                idv = plsc.bitcast(bb0[hw, pl.ds(hh, L)], jnp.int32)
                plsc.store_scatter(fidxb, [idv, lv],
                                   jnp.full((L,), hw, jnp.int32))

        def gather_pass(buf, obuf):
            src = bbs[buf]
            dst = obs[obuf]

            @plsc.parallel_loop(0, NH)
            def _gather(h):
                hh = h * L
                lv = lane + hh
                for p in range(P):
                    fv = fidxb[p + 1, pl.ds(hh, L)]
                    gv = plsc.load_gather(src, [fv, lv])
                    dst[p, 0, pl.ds(hh, L)] = gv

        def pair(k, _):
            c0 = 2 * k
            in_dma(c0, 1).wait()

            @pl.when(k > 0)
            def _():
                out_dma(c0 - 2, 0).wait()
            in_dma(c0 + 1, 0).start()
            gather_pass(1, 0)
            out_dma(c0, 0).start()

            c1 = 2 * k + 1
            in_dma(c1, 0).wait()

            @pl.when(k > 0)
            def _():
                out_dma(c1 - 2, 1).wait()
            in_dma(c1 + 1, 1).start()
            gather_pass(0, 1)
            out_dma(c1, 1).start()
            return 0
        lax.fori_loop(0, (C - 1) // 2, pair, 0)

        in_dma(C - 1, 1).wait()
        out_dma(C - 3, 0).wait()
        gather_pass(1, 0)
        out_dma(C - 1, 0).start()

        out_dma(C - 2, 1).wait()

        @plsc.parallel_loop(0, NH)
        def _fill0(h):
            for p in range(P):
                ob1[p, 0, pl.ds(h * L, L)] = zero16

        for c in range(C, L - 1):
            pltpu.sync_copy(ob1, raw_hbm.at[:, pl.ds(c, 1), pl.ds(col0, WB)])

        out_dma(C - 1, 0).wait()

        @plsc.parallel_loop(0, NH)
        def _fill1(h):
            for p in range(P):
                ob0[p, 0, pl.ds(h * L, L)] = zero16 + 1.0

        pltpu.sync_copy(ob0, raw_hbm.at[:, pl.ds(L - 1, 1), pl.ds(col0, WB)])

    return _sc_extract


def _tc_project(w_ref, x_ref, o_ref):
    o_ref[...] = jnp.dot(w_ref[...], x_ref[...],
                         preferred_element_type=jnp.float32)


def _tc_project_alias(w_ref, x_ref, y_ref, o_ref):
    del y_ref
    o_ref[...] = jnp.dot(w_ref[...], x_ref[...],
                         preferred_element_type=jnp.float32)


def kernel(full_board_vector, piece_ids, proj_w, proj_b):
    ids_t = jax.lax.bitcast_convert_type(
        piece_ids.transpose(1, 2, 0).reshape(HW, B), jnp.float32)
    board_t = full_board_vector.transpose(1, 2, 3, 0).reshape(C * HW, B)

    sc_scratch = [
        pltpu.VMEM((HW + 1, WB), jnp.float32),
        pltpu.VMEM((HW + 1, WB), jnp.float32),
        pltpu.VMEM((P + 1, WB), jnp.int32),
        pltpu.VMEM((P, 1, WB), jnp.float32),
        pltpu.VMEM((P, 1, WB), jnp.float32),
        pltpu.SemaphoreType.DMA,
        pltpu.SemaphoreType.DMA,
        pltpu.SemaphoreType.DMA,
        pltpu.SemaphoreType.DMA,
    ]
    mesh = plsc.VectorSubcoreMesh(core_axis_name="c", subcore_axis_name="s",
                                  num_cores=NC, num_subcores=NS)
    raws = []
    for s in range(S):
        sc_call = pl.kernel(
            _make_sc_body(s),
            out_type=jax.ShapeDtypeStruct((P, L, BS), jnp.float32),
            mesh=mesh,
            compiler_params=pltpu.CompilerParams(needs_layout_passes=False),
            scratch_types=sc_scratch,
        )
        raws.append(sc_call(ids_t, board_t).reshape(P * L, BS))

    wb = jnp.zeros((OUT, L), jnp.float32).at[:, :C].set(proj_w)
    wb = wb.at[:, L - 1].set(proj_b)
    w16 = jnp.kron(jnp.eye(L, dtype=jnp.float32), wb)

    NB = 2048
    nbs = BS // NB
    y = None
    for s in range(S):
        w_spec = pl.BlockSpec((L * OUT, L * L), lambda g, nb: (0, 0))
        x_spec = pl.BlockSpec((L * L, NB), lambda g, nb: (g, nb))
        o_spec = pl.BlockSpec((L * OUT, NB),
                              lambda g, nb, s=s: (g, nb + s * nbs))
        if y is None:
            y = pl.pallas_call(
                _tc_project,
                grid=(2, nbs),
                in_specs=[w_spec, x_spec],
                out_specs=o_spec,
                out_shape=jax.ShapeDtypeStruct((P * OUT, B), jnp.float32),
            )(w16, raws[s])
        else:
            y = pl.pallas_call(
                _tc_project_alias,
                grid=(2, nbs),
                in_specs=[w_spec, x_spec,
                          pl.BlockSpec(memory_space=pl.ANY)],
                out_specs=o_spec,
                out_shape=jax.ShapeDtypeStruct((P * OUT, B), jnp.float32),
                input_output_aliases={2: 0},
            )(w16, raws[s], y)
    return y.reshape(P, OUT, B).transpose(2, 0, 1)


# speedup vs baseline: 1.3260x; 1.1130x over previous
import jax
import jax.numpy as jnp
from jax import lax
from jax.experimental import pallas as pl
from jax.experimental.pallas import tpu as pltpu
from jax.experimental.pallas import tpu_sc as plsc

B = 16384
C = 11
HW = 64
P = 32
OUT = 64
L = 16
NC = 2
NS = 16
NW = NC * NS
WB = B // NW
NH = WB // L


def _sc_extract(ids_hbm, board_hbm, raw_hbm,
                bb0, bb1, fidxb, ob0, ob1, si0, si1, so0, so1):
    cid = lax.axis_index("c")
    sid = lax.axis_index("s")
    wid = sid * NC + cid
    col0 = wid * WB
    lane = lax.iota(jnp.int32, L)
    zero16 = jnp.zeros((L,), jnp.float32)
    sent = jnp.full((L,), HW, jnp.int32)

    bbs = (bb0, bb1)
    obs = (ob0, ob1)
    sis = (si0, si1)
    sos = (so0, so1)

    def in_dma(c, buf):
        return pltpu.make_async_copy(
            board_hbm.at[pl.ds(c * HW, HW), pl.ds(col0, WB)],
            bbs[buf].at[pl.ds(0, HW)], sis[buf])

    def out_dma(c, buf):
        return pltpu.make_async_copy(
            obs[buf], raw_hbm.at[:, pl.ds(c, 1), pl.ds(col0, WB)], sos[buf])

    ids_cp = pltpu.make_async_copy(ids_hbm.at[:, pl.ds(col0, WB)],
                                   bb0.at[pl.ds(0, HW)], si0)
    ids_cp.start()
    in_dma(0, 1).start()

    @plsc.parallel_loop(0, NH)
    def _init(h):
        hh = h * L
        bb0[HW, pl.ds(hh, L)] = zero16
        bb1[HW, pl.ds(hh, L)] = zero16
        for p in range(P + 1):
            fidxb[p, pl.ds(hh, L)] = sent

    ids_cp.wait()

    @plsc.parallel_loop(0, NH)
    def _scan(h):
        hh = h * L
        lv = lane + hh
        for hw in range(HW - 1, -1, -1):
            idv = plsc.bitcast(bb0[hw, pl.ds(hh, L)], jnp.int32)
            plsc.store_scatter(fidxb, [idv, lv], jnp.full((L,), hw, jnp.int32))

    def gather_pass(buf, obuf):
        src = bbs[buf]
        dst = obs[obuf]

        @plsc.parallel_loop(0, NH)
        def _gather(h):
            hh = h * L
            lv = lane + hh
            for p in range(P):
                fv = fidxb[p + 1, pl.ds(hh, L)]
                gv = plsc.load_gather(src, [fv, lv])
                dst[p, 0, pl.ds(hh, L)] = gv

    def pair(k, _):
        c0 = 2 * k
        in_dma(c0, 1).wait()

        @pl.when(k > 0)
        def _():
            out_dma(c0 - 2, 0).wait()
        in_dma(c0 + 1, 0).start()
        gather_pass(1, 0)
        out_dma(c0, 0).start()

        c1 = 2 * k + 1
        in_dma(c1, 0).wait()

        @pl.when(k > 0)
        def _():
            out_dma(c1 - 2, 1).wait()
        in_dma(c1 + 1, 1).start()
        gather_pass(0, 1)
        out_dma(c1, 1).start()
        return 0
    lax.fori_loop(0, (C - 1) // 2, pair, 0)

    in_dma(C - 1, 1).wait()
    out_dma(C - 3, 0).wait()
    gather_pass(1, 0)
    out_dma(C - 1, 0).start()

    out_dma(C - 2, 1).wait()

    @plsc.parallel_loop(0, NH)
    def _fill0(h):
        for p in range(P):
            ob1[p, 0, pl.ds(h * L, L)] = zero16

    for c in range(C, L - 1):
        pltpu.sync_copy(ob1, raw_hbm.at[:, pl.ds(c, 1), pl.ds(col0, WB)])

    out_dma(C - 1, 0).wait()

    @plsc.parallel_loop(0, NH)
    def _fill1(h):
        for p in range(P):
            ob0[p, 0, pl.ds(h * L, L)] = zero16 + 1.0

    pltpu.sync_copy(ob0, raw_hbm.at[:, pl.ds(L - 1, 1), pl.ds(col0, WB)])


def _tc_project(w_ref, x_ref, o_ref):
    o_ref[...] = jnp.dot(w_ref[...], x_ref[...],
                         preferred_element_type=jnp.float32)


def kernel(full_board_vector, piece_ids, proj_w, proj_b):
    ids_t = jax.lax.bitcast_convert_type(
        piece_ids.transpose(1, 2, 0).reshape(HW, B), jnp.float32)
    board_t = full_board_vector.transpose(1, 2, 3, 0).reshape(C * HW, B)

    sc_call = pl.kernel(
        _sc_extract,
        out_type=jax.ShapeDtypeStruct((P, L, B), jnp.float32),
        mesh=plsc.VectorSubcoreMesh(core_axis_name="c", subcore_axis_name="s",
                                    num_cores=NC, num_subcores=NS),
        compiler_params=pltpu.CompilerParams(needs_layout_passes=False),
        scratch_types=[
            pltpu.VMEM((HW + 1, WB), jnp.float32),
            pltpu.VMEM((HW + 1, WB), jnp.float32),
            pltpu.VMEM((P + 1, WB), jnp.int32),
            pltpu.VMEM((P, 1, WB), jnp.float32),
            pltpu.VMEM((P, 1, WB), jnp.float32),
            pltpu.SemaphoreType.DMA,
            pltpu.SemaphoreType.DMA,
            pltpu.SemaphoreType.DMA,
            pltpu.SemaphoreType.DMA,
        ],
    )
    raw_t = sc_call(ids_t, board_t).reshape(P * L, B)

    wb = jnp.zeros((OUT, L), jnp.float32).at[:, :C].set(proj_w)
    wb = wb.at[:, L - 1].set(proj_b)
    w16 = jnp.kron(jnp.eye(L, dtype=jnp.float32), wb)

    NB = 4096
    y = pl.pallas_call(
        _tc_project,
        grid=(2, B // NB),
        in_specs=[
            pl.BlockSpec((L * OUT, L * L), lambda g, nb: (0, 0)),
            pl.BlockSpec((L * L, NB), lambda g, nb: (g, nb)),
        ],
        out_specs=pl.BlockSpec((L * OUT, NB), lambda g, nb: (g, nb)),
        out_shape=jax.ShapeDtypeStruct((P * OUT, B), jnp.float32),
    )(w16, raw_t)
    return y.reshape(P, OUT, B).transpose(2, 0, 1)
